# baseline (speedup 1.0000x reference)
import functools

import jax
import jax.numpy as jnp
from jax import lax
from jax.experimental import pallas as pl
from jax.experimental.pallas import tpu as pltpu
from jax.experimental.pallas import tpu_sc as plsc

N = 100000
D_IN = 308
D = 100
DP = 128
DPK = DP // 2
NW = 32
WPW = 3200
NP = NW * WPW
CH = 16
NCHUNK = WPW // CH
IPC = CH * 5
NNF = N // 10


def _proj_body(vv_ref, wa_ref, ba_ref, uu_ref):
    acc = jnp.dot(vv_ref[...], wa_ref[...], preferred_element_type=jnp.float32)
    uu_ref[...] = jnp.maximum(acc + ba_ref[...], 0.0).astype(jnp.bfloat16)


def _project(vv, wa_p, ba_p):
    bn = 1000
    return pl.pallas_call(
        _proj_body,
        grid=(N // bn,),
        in_specs=[
            pl.BlockSpec((bn, D_IN), lambda i: (i, 0)),
            pl.BlockSpec((D_IN, DP), lambda i: (0, 0)),
            pl.BlockSpec((1, DP), lambda i: (0, 0)),
        ],
        out_specs=pl.BlockSpec((bn, DP), lambda i: (i, 0)),
        out_shape=jax.ShapeDtypeStruct((N, DP), jnp.bfloat16),
    )(vv, wa_p, ba_p)


def _gather_body(uu, idx3d, inf, bb, idx_v, inf_v, mask_v, rows0, rows1,
                 bb_buf, sem0, sem1):
    cid = lax.axis_index("c")
    sid = lax.axis_index("s")
    wid = sid * 2 + cid
    wb = wid * WPW

    pltpu.sync_copy(idx3d.at[wid], idx_v)
    pltpu.sync_copy(inf, inf_v)

    ones = jnp.ones((16,), jnp.float32)
    zeros = jnp.zeros((16,), jnp.float32)

    @pl.loop(0, WPW // 16)
    def _init_mask(i):
        mask_v[pl.ds(i * 16, 16)] = ones

    @pl.loop(0, NNF // 16)
    def _scatter_mask(i):
        v = inf_v[pl.ds(i * 16, 16)]
        loc = v - wb
        inb = (loc >= 0) & (loc < WPW)
        locc = jnp.clip(loc, 0, WPW - 1)
        plsc.store_scatter(mask_v, [locc], zeros, mask=inb)

    def issue(c, rbuf, sem):
        pltpu.async_copy(uu.at[idx_v.at[c]], rbuf, sem)

    def wait(c, rbuf, sem):
        pltpu.make_async_copy(uu.at[idx_v.at[c]], rbuf, sem).wait()

    issue(0, rows0, sem0)
    issue(1, rows1, sem1)

    bf = jnp.bfloat16

    @pl.loop(0, NCHUNK, step=2)
    def _chunks(c0):
        for b in range(2):
            c = c0 + b
            rbuf = rows0 if b == 0 else rows1
            sem = sem0 if b == 0 else sem1
            wait(c, rbuf, sem)
            mvv = mask_v[pl.ds(c * CH, 16)]
            for j in range(CH):
                mvf = jnp.full((16,), mvv[j], jnp.float32)
                mv = plsc.pack(mvf, mvf, format=plsc.PackFormat.INTERLEAVED)
                r = j * 5
                for t in range(DPK // 16):
                    s = pl.ds(t * 16, 16)
                    acc = (plsc.bitcast(rbuf[r, s], bf)
                           + plsc.bitcast(rbuf[r + 1, s], bf)
                           + plsc.bitcast(rbuf[r + 2, s], bf)
                           + plsc.bitcast(rbuf[r + 3, s], bf)
                           + plsc.bitcast(rbuf[r + 4, s], bf))
                    bb_buf[j, s] = plsc.bitcast(acc * mv, jnp.float32)
            pltpu.sync_copy(bb_buf, bb.at[pl.ds(wb + c * CH, CH)])

            @pl.when(c + 2 < NCHUNK)
            def _next():
                issue(c + 2, rbuf, sem)


def _gather_sum(uu_packed, idx3d, inf):
    mesh = plsc.VectorSubcoreMesh(core_axis_name="c", subcore_axis_name="s",
                                  num_cores=2, num_subcores=16)
    kern = pl.kernel(
        _gather_body,
        out_type=jax.ShapeDtypeStruct((NP, DPK), jnp.float32),
        mesh=mesh,
        compiler_params=pltpu.CompilerParams(needs_layout_passes=False,
                                             use_tc_tiling_on_sc=False),
        scratch_types=[
            pltpu.VMEM((NCHUNK, IPC), jnp.int32),
            pltpu.VMEM((NNF,), jnp.int32),
            pltpu.VMEM((WPW,), jnp.float32),
            pltpu.VMEM((IPC, DPK), jnp.float32),
            pltpu.VMEM((IPC, DPK), jnp.float32),
            pltpu.VMEM((CH, DPK), jnp.float32),
            pltpu.SemaphoreType.DMA,
            pltpu.SemaphoreType.DMA,
        ],
    )
    return kern(uu_packed, idx3d, inf)


def _mlp_body(bb_ref, w1_ref, b1_ref, wo_ref, bo_ref, w3_ref, st_ref):
    hid = jnp.tanh(
        jnp.dot(bb_ref[...], w1_ref[...], preferred_element_type=jnp.float32)
        + b1_ref[...])
    oo = jnp.tanh(
        jnp.dot(hid.astype(jnp.bfloat16), wo_ref[...],
                preferred_element_type=jnp.float32)
        + bo_ref[...])
    st_ref[...] = lax.dot_general(w3_ref[...], oo, (((1,), (1,)), ((), ())),
                                  preferred_element_type=jnp.float32)


def _mlp_scores(bb, w1_p, b1_p, wo_p, bo_p, w3_p):
    bn = 1024
    return pl.pallas_call(
        _mlp_body,
        grid=(NP // bn,),
        in_specs=[
            pl.BlockSpec((bn, DP), lambda i: (i, 0)),
            pl.BlockSpec((DP, DP), lambda i: (0, 0)),
            pl.BlockSpec((1, DP), lambda i: (0, 0)),
            pl.BlockSpec((DP, DP), lambda i: (0, 0)),
            pl.BlockSpec((1, DP), lambda i: (0, 0)),
            pl.BlockSpec((8, DP), lambda i: (0, 0)),
        ],
        out_specs=pl.BlockSpec((8, bn), lambda i: (0, i)),
        out_shape=jax.ShapeDtypeStruct((8, NP), jnp.float32),
    )(bb, w1_p, b1_p, wo_p, bo_p, w3_p)


def _softmax_body(st_ref, out_ref):
    x = st_ref[...]
    col = lax.broadcasted_iota(jnp.int32, (8, NP), 1)
    valid = col < N
    xm = jnp.where(valid, x, -jnp.inf)
    mx = jnp.max(xm, axis=1, keepdims=True)
    e = jnp.where(valid, jnp.exp(x - mx), 0.0)
    ssum = jnp.sum(e, axis=1, keepdims=True)
    out_ref[...] = e / ssum


def _softmax(st):
    return pl.pallas_call(
        _softmax_body,
        grid=(1,),
        in_specs=[pl.BlockSpec((8, NP), lambda i: (0, 0))],
        out_specs=pl.BlockSpec((8, NP), lambda i: (0, 0)),
        out_shape=jax.ShapeDtypeStruct((8, NP), jnp.float32),
    )(st)


def kernel(indices, indices_not_found, vv, num_words, W_A, b_A, W_b1, b_b1,
           W_bo, b_bo, W_bh, b_bh, W_D, b_D, w_r, b_r, w_c, b_c, w_ce, b_ce):
    bf = jnp.bfloat16
    pad_d = DP - D
    wa_p = jnp.pad(W_A, ((0, 0), (0, pad_d))).astype(bf)
    ba_p = jnp.pad(b_A, (0, pad_d)).reshape(1, DP)
    uu = _project(vv.astype(bf), wa_p, ba_p)
    uu_packed = lax.bitcast_convert_type(
        uu.reshape(N, DPK, 2), jnp.float32)

    idx3d = jnp.pad(indices, ((0, NP - N), (0, 0))).reshape(NW, NCHUNK, IPC)
    bb_packed = _gather_sum(uu_packed, idx3d, indices_not_found)
    bb = lax.bitcast_convert_type(bb_packed, bf).reshape(NP, DP)

    w1_p = jnp.pad(W_b1, ((0, pad_d), (0, pad_d))).astype(bf)
    b1_p = jnp.pad(b_b1, (0, pad_d)).reshape(1, DP)
    wo_p = jnp.pad(W_bo, ((0, pad_d), (0, pad_d))).astype(bf)
    bo_p = jnp.pad(b_bo, (0, pad_d)).reshape(1, DP)
    w3 = jnp.stack([w_r, w_c, w_ce]).astype(jnp.float32)
    w3_p = jnp.pad(w3, ((0, 5), (0, pad_d)))
    st = _mlp_scores(bb, w1_p, b1_p, wo_p, bo_p, w3_p)

    sm = _softmax(st)
    return (sm[0, :N], sm[1, :N], sm[2, :N])


# speedup vs baseline: 1.7287x; 1.7287x over previous
import functools

import jax
import jax.numpy as jnp
from jax import lax
from jax.experimental import pallas as pl
from jax.experimental.pallas import tpu as pltpu
from jax.experimental.pallas import tpu_sc as plsc

N = 100000
D_IN = 308
D = 100
DP = 128
DPK = DP // 2
NW = 32
WPW = 3200
NP = NW * WPW
CH = 16
NCHUNK = WPW // CH
IPC = CH * 5
NNF = N // 10


def _rne_hi16(u):
    return (u + 0x7FFF + ((u >> 16) & 1)) & jnp.uint32(0xFFFF0000)


def _proj_body(vv_ref, wa_ref, ba_ref, uu_ref):
    x = vv_ref[...].astype(jnp.bfloat16)
    acc = jnp.dot(x, wa_ref[...], preferred_element_type=jnp.float32)
    uu = jnp.maximum(acc + ba_ref[...], 0.0)
    a = lax.bitcast_convert_type(uu[:, :DPK], jnp.uint32)
    b = lax.bitcast_convert_type(uu[:, DPK:], jnp.uint32)
    packed = _rne_hi16(a) | (_rne_hi16(b) >> 16)
    uu_ref[...] = lax.bitcast_convert_type(packed, jnp.float32)


def _project(vv, wa_p, ba_p):
    bn = 1000
    return pl.pallas_call(
        _proj_body,
        grid=(N // bn,),
        in_specs=[
            pl.BlockSpec((bn, D_IN), lambda i: (i, 0)),
            pl.BlockSpec((D_IN, DP), lambda i: (0, 0)),
            pl.BlockSpec((1, DP), lambda i: (0, 0)),
        ],
        out_specs=pl.BlockSpec((bn, DPK), lambda i: (i, 0)),
        out_shape=jax.ShapeDtypeStruct((N, DPK), jnp.float32),
    )(vv, wa_p, ba_p)


def _gather_body(uu, idx3d, inf, bb, idx_v, inf_v, mask_v, rows0, rows1,
                 bb_buf, sem0, sem1):
    cid = lax.axis_index("c")
    sid = lax.axis_index("s")
    wid = sid * 2 + cid
    wb = wid * WPW

    pltpu.sync_copy(idx3d.at[wid], idx_v)
    pltpu.sync_copy(inf, inf_v)

    ones = jnp.ones((16,), jnp.float32)
    zeros = jnp.zeros((16,), jnp.float32)

    @pl.loop(0, WPW // 16)
    def _init_mask(i):
        mask_v[pl.ds(i * 16, 16)] = ones

    @pl.loop(0, NNF // 16)
    def _scatter_mask(i):
        v = inf_v[pl.ds(i * 16, 16)]
        loc = v - wb
        inb = (loc >= 0) & (loc < WPW)
        locc = jnp.clip(loc, 0, WPW - 1)
        plsc.store_scatter(mask_v, [locc], zeros, mask=inb)

    def issue(c, rbuf, sem):
        pltpu.async_copy(uu.at[idx_v.at[c]], rbuf, sem)

    def wait(c, rbuf, sem):
        pltpu.make_async_copy(uu.at[idx_v.at[c]], rbuf, sem).wait()

    issue(0, rows0, sem0)
    issue(1, rows1, sem1)

    bf = jnp.bfloat16

    @pl.loop(0, NCHUNK, step=2)
    def _chunks(c0):
        for b in range(2):
            c = c0 + b
            rbuf = rows0 if b == 0 else rows1
            sem = sem0 if b == 0 else sem1
            wait(c, rbuf, sem)
            mvv = mask_v[pl.ds(c * CH, 16)]
            for j in range(CH):
                mvf = jnp.full((16,), mvv[j], jnp.float32)
                mv = plsc.pack(mvf, mvf, format=plsc.PackFormat.INTERLEAVED)
                r = j * 5
                for t in range(DPK // 16):
                    s = pl.ds(t * 16, 16)
                    acc = (plsc.bitcast(rbuf[r, s], bf)
                           + plsc.bitcast(rbuf[r + 1, s], bf)
                           + plsc.bitcast(rbuf[r + 2, s], bf)
                           + plsc.bitcast(rbuf[r + 3, s], bf)
                           + plsc.bitcast(rbuf[r + 4, s], bf))
                    bb_buf[j, s] = plsc.bitcast(acc * mv, jnp.float32)
            pltpu.sync_copy(bb_buf, bb.at[pl.ds(wb + c * CH, CH)])

            @pl.when(c + 2 < NCHUNK)
            def _next():
                issue(c + 2, rbuf, sem)


def _gather_sum(uu_packed, idx3d, inf):
    mesh = plsc.VectorSubcoreMesh(core_axis_name="c", subcore_axis_name="s",
                                  num_cores=2, num_subcores=16)
    kern = pl.kernel(
        _gather_body,
        out_type=jax.ShapeDtypeStruct((NP, DPK), jnp.float32),
        mesh=mesh,
        compiler_params=pltpu.CompilerParams(needs_layout_passes=False,
                                             use_tc_tiling_on_sc=False),
        scratch_types=[
            pltpu.VMEM((NCHUNK, IPC), jnp.int32),
            pltpu.VMEM((NNF,), jnp.int32),
            pltpu.VMEM((WPW,), jnp.float32),
            pltpu.VMEM((IPC, DPK), jnp.float32),
            pltpu.VMEM((IPC, DPK), jnp.float32),
            pltpu.VMEM((CH, DPK), jnp.float32),
            pltpu.SemaphoreType.DMA,
            pltpu.SemaphoreType.DMA,
        ],
    )
    return kern(uu_packed, idx3d, inf)


def _mlp_body(bb_ref, w1_ref, b1_ref, wo_ref, bo_ref, w3_ref, st_ref):
    pu = lax.bitcast_convert_type(bb_ref[...], jnp.uint32)
    hi = lax.bitcast_convert_type(pu & jnp.uint32(0xFFFF0000), jnp.float32)
    lo = lax.bitcast_convert_type(pu << 16, jnp.float32)
    bb = jnp.concatenate([hi, lo], axis=1).astype(jnp.bfloat16)
    hid = jnp.tanh(
        jnp.dot(bb, w1_ref[...], preferred_element_type=jnp.float32)
        + b1_ref[...])
    oo = jnp.tanh(
        jnp.dot(hid.astype(jnp.bfloat16), wo_ref[...],
                preferred_element_type=jnp.float32)
        + bo_ref[...])
    st_ref[...] = lax.dot_general(w3_ref[...], oo, (((1,), (1,)), ((), ())),
                                  preferred_element_type=jnp.float32)


def _mlp_scores(bb, w1_p, b1_p, wo_p, bo_p, w3_p):
    bn = 1024
    return pl.pallas_call(
        _mlp_body,
        grid=(NP // bn,),
        in_specs=[
            pl.BlockSpec((bn, DPK), lambda i: (i, 0)),
            pl.BlockSpec((DP, DP), lambda i: (0, 0)),
            pl.BlockSpec((1, DP), lambda i: (0, 0)),
            pl.BlockSpec((DP, DP), lambda i: (0, 0)),
            pl.BlockSpec((1, DP), lambda i: (0, 0)),
            pl.BlockSpec((8, DP), lambda i: (0, 0)),
        ],
        out_specs=pl.BlockSpec((8, bn), lambda i: (0, i)),
        out_shape=jax.ShapeDtypeStruct((8, NP), jnp.float32),
    )(bb, w1_p, b1_p, wo_p, bo_p, w3_p)


def _softmax_body(st_ref, out_ref):
    x = st_ref[...]
    col = lax.broadcasted_iota(jnp.int32, (8, NP), 1)
    valid = col < N
    xm = jnp.where(valid, x, -jnp.inf)
    mx = jnp.max(xm, axis=1, keepdims=True)
    e = jnp.where(valid, jnp.exp(x - mx), 0.0)
    ssum = jnp.sum(e, axis=1, keepdims=True)
    out_ref[...] = e / ssum


def _softmax(st):
    return pl.pallas_call(
        _softmax_body,
        grid=(1,),
        in_specs=[pl.BlockSpec((8, NP), lambda i: (0, 0))],
        out_specs=pl.BlockSpec((8, NP), lambda i: (0, 0)),
        out_shape=jax.ShapeDtypeStruct((8, NP), jnp.float32),
    )(st)


def kernel(indices, indices_not_found, vv, num_words, W_A, b_A, W_b1, b_b1,
           W_bo, b_bo, W_bh, b_bh, W_D, b_D, w_r, b_r, w_c, b_c, w_ce, b_ce):
    bf = jnp.bfloat16
    pad_d = DP - D
    wa_p = jnp.pad(W_A, ((0, 0), (0, pad_d))).astype(bf)
    ba_p = jnp.pad(b_A, (0, pad_d)).reshape(1, DP)
    uu_packed = _project(vv, wa_p, ba_p)

    idx3d = jnp.pad(indices, ((0, NP - N), (0, 0))).reshape(NW, NCHUNK, IPC)
    bb = _gather_sum(uu_packed, idx3d, indices_not_found)

    w1_p = jnp.pad(W_b1, ((0, pad_d), (0, pad_d))).astype(bf)
    b1_p = jnp.pad(b_b1, (0, pad_d)).reshape(1, DP)
    wo_p = jnp.pad(W_bo, ((0, pad_d), (0, pad_d))).astype(bf)
    bo_p = jnp.pad(b_bo, (0, pad_d)).reshape(1, DP)
    w3 = jnp.stack([w_r, w_c, w_ce]).astype(jnp.float32)
    w3_p = jnp.pad(w3, ((0, 5), (0, pad_d)))
    st = _mlp_scores(bb, w1_p, b1_p, wo_p, bo_p, w3_p)

    sm = _softmax(st)
    return (sm[0, :N], sm[1, :N], sm[2, :N])


# speedup vs baseline: 1.7867x; 1.0336x over previous
import functools

import jax
import jax.numpy as jnp
from jax import lax
from jax.experimental import pallas as pl
from jax.experimental.pallas import tpu as pltpu
from jax.experimental.pallas import tpu_sc as plsc

N = 100000
D_IN = 308
D = 100
DP = 128
DPK = DP // 2
NW = 32
WPW = 3200
NP = NW * WPW
CH = 16
NCHUNK = WPW // CH
IPC = CH * 5
NNF = N // 10


def _rne_hi16(u):
    return (u + 0x7FFF + ((u >> 16) & 1)) & jnp.uint32(0xFFFF0000)


def _proj_body(vv_ref, wa_ref, ba_ref, uu_ref):
    x = vv_ref[...].astype(jnp.bfloat16)
    acc = jnp.dot(x, wa_ref[...], preferred_element_type=jnp.float32)
    uu = jnp.maximum(acc + ba_ref[...], 0.0)
    a = lax.bitcast_convert_type(uu[:, :DPK], jnp.uint32)
    b = lax.bitcast_convert_type(uu[:, DPK:], jnp.uint32)
    packed = _rne_hi16(a) | (_rne_hi16(b) >> 16)
    uu_ref[...] = lax.bitcast_convert_type(packed, jnp.float32)


def _project(vv, wa_p, ba_p):
    bn = 1000
    return pl.pallas_call(
        _proj_body,
        grid=(N // bn,),
        in_specs=[
            pl.BlockSpec((bn, D_IN), lambda i: (i, 0)),
            pl.BlockSpec((D_IN, DP), lambda i: (0, 0)),
            pl.BlockSpec((1, DP), lambda i: (0, 0)),
        ],
        out_specs=pl.BlockSpec((bn, DPK), lambda i: (i, 0)),
        out_shape=jax.ShapeDtypeStruct((N, DPK), jnp.float32),
    )(vv, wa_p, ba_p)


def _gather_body(uu, idx_f, padidx, inf, bb, idx_v, inf_v, mask_v, rows0,
                 rows1, bb_buf, sem0, sem1):
    cid = lax.axis_index("c")
    sid = lax.axis_index("s")
    wid = sid * 2 + cid
    wb = wid * WPW

    @pl.when(wid < NW - 1)
    def _stage_full():
        pltpu.sync_copy(idx_f.at[pl.ds(wb * 5, WPW * 5)], idx_v)

    @pl.when(wid == NW - 1)
    def _stage_tail():
        tail5 = (N - (NW - 1) * WPW) * 5
        pltpu.sync_copy(idx_f.at[pl.ds((NW - 1) * WPW * 5, tail5)],
                        idx_v.at[pl.ds(0, tail5)])
        pltpu.sync_copy(padidx, idx_v.at[pl.ds(tail5, (NP - N) * 5)])

    pltpu.sync_copy(inf, inf_v)

    ones = jnp.ones((16,), jnp.float32)
    zeros = jnp.zeros((16,), jnp.float32)

    @pl.loop(0, WPW // 16)
    def _init_mask(i):
        mask_v[pl.ds(i * 16, 16)] = ones

    @pl.loop(0, NNF // 16)
    def _scatter_mask(i):
        v = inf_v[pl.ds(i * 16, 16)]
        loc = v - wb
        inb = (loc >= 0) & (loc < WPW)
        locc = jnp.clip(loc, 0, WPW - 1)
        plsc.store_scatter(mask_v, [locc], zeros, mask=inb)

    def issue(c, rbuf, sem):
        pltpu.async_copy(uu.at[idx_v.at[pl.ds(c * IPC, IPC)]], rbuf, sem)

    def wait(c, rbuf, sem):
        pltpu.make_async_copy(uu.at[idx_v.at[pl.ds(c * IPC, IPC)]], rbuf,
                              sem).wait()

    issue(0, rows0, sem0)
    issue(1, rows1, sem1)

    bf = jnp.bfloat16

    @pl.loop(0, NCHUNK, step=2)
    def _chunks(c0):
        for b in range(2):
            c = c0 + b
            rbuf = rows0 if b == 0 else rows1
            sem = sem0 if b == 0 else sem1
            wait(c, rbuf, sem)
            mvv = mask_v[pl.ds(c * CH, 16)]
            for j in range(CH):
                mvf = jnp.full((16,), mvv[j], jnp.float32)
                mv = plsc.pack(mvf, mvf, format=plsc.PackFormat.INTERLEAVED)
                for t in range(DPK // 16):
                    s = pl.ds(t * 16, 16)
                    r = j * 5
                    acc = (plsc.bitcast(rbuf[r, s], bf)
                           + plsc.bitcast(rbuf[r + 1, s], bf)
                           + plsc.bitcast(rbuf[r + 2, s], bf)
                           + plsc.bitcast(rbuf[r + 3, s], bf)
                           + plsc.bitcast(rbuf[r + 4, s], bf))
                    bb_buf[j, s] = plsc.bitcast(acc * mv, jnp.float32)
            pltpu.sync_copy(bb_buf, bb.at[pl.ds(wb + c * CH, CH)])

            @pl.when(c + 2 < NCHUNK)
            def _next():
                issue(c + 2, rbuf, sem)


def _gather_sum(uu_packed, idx, padidx, inf):
    mesh = plsc.VectorSubcoreMesh(core_axis_name="c", subcore_axis_name="s",
                                  num_cores=2, num_subcores=16)
    kern = pl.kernel(
        _gather_body,
        out_type=jax.ShapeDtypeStruct((NP, DPK), jnp.float32),
        mesh=mesh,
        compiler_params=pltpu.CompilerParams(needs_layout_passes=False,
                                             use_tc_tiling_on_sc=False),
        scratch_types=[
            pltpu.VMEM((WPW * 5,), jnp.int32),
            pltpu.VMEM((NNF,), jnp.int32),
            pltpu.VMEM((WPW,), jnp.float32),
            pltpu.VMEM((IPC, DPK), jnp.float32),
            pltpu.VMEM((IPC, DPK), jnp.float32),
            pltpu.VMEM((CH, DPK), jnp.float32),
            pltpu.SemaphoreType.DMA,
            pltpu.SemaphoreType.DMA,
        ],
    )
    return kern(uu_packed, idx, padidx, inf)


def _mlp_body(bb_ref, w1_ref, b1_ref, wo_ref, bo_ref, w3_ref, st_ref):
    pu = lax.bitcast_convert_type(bb_ref[...], jnp.uint32)
    hi = lax.bitcast_convert_type(pu & jnp.uint32(0xFFFF0000), jnp.float32)
    lo = lax.bitcast_convert_type(pu << 16, jnp.float32)
    bb = jnp.concatenate([hi, lo], axis=1).astype(jnp.bfloat16)
    hid = jnp.tanh(
        jnp.dot(bb, w1_ref[...], preferred_element_type=jnp.float32)
        + b1_ref[...])
    oo = jnp.tanh(
        jnp.dot(hid.astype(jnp.bfloat16), wo_ref[...],
                preferred_element_type=jnp.float32)
        + bo_ref[...])
    st_ref[...] = lax.dot_general(w3_ref[...], oo, (((1,), (1,)), ((), ())),
                                  preferred_element_type=jnp.float32)


def _mlp_scores(bb, w1_p, b1_p, wo_p, bo_p, w3_p):
    bn = 1024
    return pl.pallas_call(
        _mlp_body,
        grid=(NP // bn,),
        in_specs=[
            pl.BlockSpec((bn, DPK), lambda i: (i, 0)),
            pl.BlockSpec((DP, DP), lambda i: (0, 0)),
            pl.BlockSpec((1, DP), lambda i: (0, 0)),
            pl.BlockSpec((DP, DP), lambda i: (0, 0)),
            pl.BlockSpec((1, DP), lambda i: (0, 0)),
            pl.BlockSpec((8, DP), lambda i: (0, 0)),
        ],
        out_specs=pl.BlockSpec((8, bn), lambda i: (0, i)),
        out_shape=jax.ShapeDtypeStruct((8, NP), jnp.float32),
    )(bb, w1_p, b1_p, wo_p, bo_p, w3_p)


def _softmax_body(st_ref, out_ref):
    x = st_ref[...]
    col = lax.broadcasted_iota(jnp.int32, (8, NP), 1)
    valid = col < N
    xm = jnp.where(valid, x, -jnp.inf)
    mx = jnp.max(xm, axis=1, keepdims=True)
    e = jnp.where(valid, jnp.exp(x - mx), 0.0)
    ssum = jnp.sum(e, axis=1, keepdims=True)
    out_ref[...] = e / ssum


def _softmax(st):
    return pl.pallas_call(
        _softmax_body,
        grid=(1,),
        in_specs=[pl.BlockSpec((8, NP), lambda i: (0, 0))],
        out_specs=pl.BlockSpec((8, NP), lambda i: (0, 0)),
        out_shape=jax.ShapeDtypeStruct((8, NP), jnp.float32),
    )(st)


def kernel(indices, indices_not_found, vv, num_words, W_A, b_A, W_b1, b_b1,
           W_bo, b_bo, W_bh, b_bh, W_D, b_D, w_r, b_r, w_c, b_c, w_ce, b_ce):
    bf = jnp.bfloat16
    pad_d = DP - D
    wa_p = jnp.pad(W_A, ((0, 0), (0, pad_d))).astype(bf)
    ba_p = jnp.pad(b_A, (0, pad_d)).reshape(1, DP)
    uu_packed = _project(vv, wa_p, ba_p)

    padidx = jnp.zeros(((NP - N) * 5,), jnp.int32)
    bb = _gather_sum(uu_packed, indices.reshape(N * 5), padidx,
                     indices_not_found)

    w1_p = jnp.pad(W_b1, ((0, pad_d), (0, pad_d))).astype(bf)
    b1_p = jnp.pad(b_b1, (0, pad_d)).reshape(1, DP)
    wo_p = jnp.pad(W_bo, ((0, pad_d), (0, pad_d))).astype(bf)
    bo_p = jnp.pad(b_bo, (0, pad_d)).reshape(1, DP)
    w3 = jnp.stack([w_r, w_c, w_ce]).astype(jnp.float32)
    w3_p = jnp.pad(w3, ((0, 5), (0, pad_d)))
    st = _mlp_scores(bb, w1_p, b1_p, wo_p, bo_p, w3_p)

    sm = _softmax(st)
    return (sm[0, :N], sm[1, :N], sm[2, :N])


# speedup vs baseline: 1.8893x; 1.0574x over previous
import functools

import jax
import jax.numpy as jnp
from jax import lax
from jax.experimental import pallas as pl
from jax.experimental.pallas import tpu as pltpu
from jax.experimental.pallas import tpu_sc as plsc

N = 100000
D_IN = 308
D = 100
DP = 128
DPK = DP // 2
NW = 32
WPW = 3200
NP = NW * WPW
CH = 16
NCHUNK = WPW // CH
IPC = CH * 5
NNF = N // 10


def _rne_hi16(u):
    return (u + 0x7FFF + ((u >> 16) & 1)) & jnp.uint32(0xFFFF0000)


def _proj_body(vv_ref, wa_ref, ba_ref, uu_ref):
    x = vv_ref[...].astype(jnp.bfloat16)
    acc = jnp.dot(x, wa_ref[...], preferred_element_type=jnp.float32)
    uu = jnp.maximum(acc + ba_ref[...], 0.0)
    a = lax.bitcast_convert_type(uu[:, :DPK], jnp.uint32)
    b = lax.bitcast_convert_type(uu[:, DPK:], jnp.uint32)
    packed = _rne_hi16(a) | (_rne_hi16(b) >> 16)
    uu_ref[...] = lax.bitcast_convert_type(packed, jnp.float32)


def _project(vv, wa_p, ba_p):
    bn = 1000
    return pl.pallas_call(
        _proj_body,
        grid=(N // bn,),
        in_specs=[
            pl.BlockSpec((bn, D_IN), lambda i: (i, 0)),
            pl.BlockSpec((D_IN, DP), lambda i: (0, 0)),
            pl.BlockSpec((1, DP), lambda i: (0, 0)),
        ],
        out_specs=pl.BlockSpec((bn, DPK), lambda i: (i, 0)),
        out_shape=jax.ShapeDtypeStruct((N, DPK), jnp.float32),
    )(vv, wa_p, ba_p)


CF = 272
CS = 400 - CF
PAIR_W = 6400


def _gather_body(uu, idx_f, padidx, inf, bb, idx_v, inf_v, mask_v, rows0,
                 rows1, bb_buf, sem0, sem1):
    cid = lax.axis_index("c")
    sid = lax.axis_index("s")
    fast = cid == 0
    nch = jnp.where(fast, CF, CS)
    wb = sid * PAIR_W + jnp.where(fast, 0, CF * CH)
    nwords = nch * CH

    last = sid == 15
    real5 = (N - 15 * PAIR_W - 0 * CH) * 5

    @pl.when(fast & ~last)
    def _stage_f():
        pltpu.sync_copy(idx_f.at[pl.ds(wb * 5, CF * IPC)], idx_v)

    @pl.when(fast & last)
    def _stage_f_tail():
        pltpu.sync_copy(idx_f.at[pl.ds(15 * PAIR_W * 5, real5)],
                        idx_v.at[pl.ds(0, real5)])
        pltpu.sync_copy(padidx.at[pl.ds(0, CF * IPC - real5)],
                        idx_v.at[pl.ds(real5, CF * IPC - real5)])

    @pl.when(~fast & ~last)
    def _stage_s():
        pltpu.sync_copy(idx_f.at[pl.ds(wb * 5, CS * IPC)],
                        idx_v.at[pl.ds(0, CS * IPC)])

    @pl.when(~fast & last)
    def _stage_s_tail():
        pltpu.sync_copy(padidx.at[pl.ds(0, CS * IPC)],
                        idx_v.at[pl.ds(0, CS * IPC)])

    pltpu.sync_copy(inf, inf_v)

    ones = jnp.ones((16,), jnp.float32)
    zeros = jnp.zeros((16,), jnp.float32)

    @pl.loop(0, CF)
    def _init_mask(i):
        mask_v[pl.ds(i * 16, 16)] = ones

    @pl.loop(0, NNF // 16)
    def _scatter_mask(i):
        v = inf_v[pl.ds(i * 16, 16)]
        loc = v - wb
        inb = (loc >= 0) & (loc < nwords)
        locc = jnp.clip(loc, 0, CF * CH - 1)
        plsc.store_scatter(mask_v, [locc], zeros, mask=inb)

    def issue(c, rbuf, sem):
        pltpu.async_copy(uu.at[idx_v.at[pl.ds(c * IPC, IPC)]], rbuf, sem)

    def wait(c, rbuf, sem):
        pltpu.make_async_copy(uu.at[idx_v.at[pl.ds(c * IPC, IPC)]], rbuf,
                              sem).wait()

    issue(0, rows0, sem0)
    issue(1, rows1, sem1)

    bf = jnp.bfloat16

    @pl.loop(0, nch, step=2)
    def _chunks(c0):
        for b in range(2):
            c = c0 + b
            rbuf = rows0 if b == 0 else rows1
            sem = sem0 if b == 0 else sem1
            wait(c, rbuf, sem)
            mvv = mask_v[pl.ds(c * CH, 16)]
            for j in range(CH):
                mvf = jnp.full((16,), mvv[j], jnp.float32)
                mv = plsc.pack(mvf, mvf, format=plsc.PackFormat.INTERLEAVED)
                for t in range(DPK // 16):
                    s = pl.ds(t * 16, 16)
                    r = j * 5
                    acc = (plsc.bitcast(rbuf[r, s], bf)
                           + plsc.bitcast(rbuf[r + 1, s], bf)
                           + plsc.bitcast(rbuf[r + 2, s], bf)
                           + plsc.bitcast(rbuf[r + 3, s], bf)
                           + plsc.bitcast(rbuf[r + 4, s], bf))
                    bb_buf[j, s] = plsc.bitcast(acc * mv, jnp.float32)
            pltpu.sync_copy(bb_buf, bb.at[pl.ds(wb + c * CH, CH)])

            @pl.when(c + 2 < nch)
            def _next():
                issue(c + 2, rbuf, sem)


def _gather_sum(uu_packed, idx, padidx, inf):
    mesh = plsc.VectorSubcoreMesh(core_axis_name="c", subcore_axis_name="s",
                                  num_cores=2, num_subcores=16)
    kern = pl.kernel(
        _gather_body,
        out_type=jax.ShapeDtypeStruct((NP, DPK), jnp.float32),
        mesh=mesh,
        compiler_params=pltpu.CompilerParams(needs_layout_passes=False,
                                             use_tc_tiling_on_sc=False),
        scratch_types=[
            pltpu.VMEM((CF * IPC,), jnp.int32),
            pltpu.VMEM((NNF,), jnp.int32),
            pltpu.VMEM((CF * CH,), jnp.float32),
            pltpu.VMEM((IPC, DPK), jnp.float32),
            pltpu.VMEM((IPC, DPK), jnp.float32),
            pltpu.VMEM((CH, DPK), jnp.float32),
            pltpu.SemaphoreType.DMA,
            pltpu.SemaphoreType.DMA,
        ],
    )
    return kern(uu_packed, idx, padidx, inf)


def _mlp_body(bb_ref, w1_ref, b1_ref, wo_ref, bo_ref, w3_ref, st_ref):
    pu = lax.bitcast_convert_type(bb_ref[...], jnp.uint32)
    hi = lax.bitcast_convert_type(pu & jnp.uint32(0xFFFF0000), jnp.float32)
    lo = lax.bitcast_convert_type(pu << 16, jnp.float32)
    bb = jnp.concatenate([hi, lo], axis=1).astype(jnp.bfloat16)
    hid = jnp.tanh(
        jnp.dot(bb, w1_ref[...], preferred_element_type=jnp.float32)
        + b1_ref[...])
    oo = jnp.tanh(
        jnp.dot(hid.astype(jnp.bfloat16), wo_ref[...],
                preferred_element_type=jnp.float32)
        + bo_ref[...])
    st_ref[...] = lax.dot_general(w3_ref[...], oo, (((1,), (1,)), ((), ())),
                                  preferred_element_type=jnp.float32)


def _mlp_scores(bb, w1_p, b1_p, wo_p, bo_p, w3_p):
    bn = 1024
    return pl.pallas_call(
        _mlp_body,
        grid=(NP // bn,),
        in_specs=[
            pl.BlockSpec((bn, DPK), lambda i: (i, 0)),
            pl.BlockSpec((DP, DP), lambda i: (0, 0)),
            pl.BlockSpec((1, DP), lambda i: (0, 0)),
            pl.BlockSpec((DP, DP), lambda i: (0, 0)),
            pl.BlockSpec((1, DP), lambda i: (0, 0)),
            pl.BlockSpec((8, DP), lambda i: (0, 0)),
        ],
        out_specs=pl.BlockSpec((8, bn), lambda i: (0, i)),
        out_shape=jax.ShapeDtypeStruct((8, NP), jnp.float32),
    )(bb, w1_p, b1_p, wo_p, bo_p, w3_p)


def _softmax_body(st_ref, out_ref):
    x = st_ref[...]
    col = lax.broadcasted_iota(jnp.int32, (8, NP), 1)
    valid = col < N
    xm = jnp.where(valid, x, -jnp.inf)
    mx = jnp.max(xm, axis=1, keepdims=True)
    e = jnp.where(valid, jnp.exp(x - mx), 0.0)
    ssum = jnp.sum(e, axis=1, keepdims=True)
    out_ref[...] = e / ssum


def _softmax(st):
    return pl.pallas_call(
        _softmax_body,
        grid=(1,),
        in_specs=[pl.BlockSpec((8, NP), lambda i: (0, 0))],
        out_specs=pl.BlockSpec((8, NP), lambda i: (0, 0)),
        out_shape=jax.ShapeDtypeStruct((8, NP), jnp.float32),
    )(st)


def kernel(indices, indices_not_found, vv, num_words, W_A, b_A, W_b1, b_b1,
           W_bo, b_bo, W_bh, b_bh, W_D, b_D, w_r, b_r, w_c, b_c, w_ce, b_ce):
    bf = jnp.bfloat16
    pad_d = DP - D
    wa_p = jnp.pad(W_A, ((0, 0), (0, pad_d))).astype(bf)
    ba_p = jnp.pad(b_A, (0, pad_d)).reshape(1, DP)
    uu_packed = _project(vv, wa_p, ba_p)

    padidx = jnp.zeros(((NP - N) * 5,), jnp.int32)
    bb = _gather_sum(uu_packed, indices.reshape(N * 5), padidx,
                     indices_not_found)

    w1_p = jnp.pad(W_b1, ((0, pad_d), (0, pad_d))).astype(bf)
    b1_p = jnp.pad(b_b1, (0, pad_d)).reshape(1, DP)
    wo_p = jnp.pad(W_bo, ((0, pad_d), (0, pad_d))).astype(bf)
    bo_p = jnp.pad(b_bo, (0, pad_d)).reshape(1, DP)
    w3 = jnp.stack([w_r, w_c, w_ce]).astype(jnp.float32)
    w3_p = jnp.pad(w3, ((0, 5), (0, pad_d)))
    st = _mlp_scores(bb, w1_p, b1_p, wo_p, bo_p, w3_p)

    sm = _softmax(st)
    return (sm[0, :N], sm[1, :N], sm[2, :N])


# speedup vs baseline: 1.9072x; 1.0095x over previous
import functools

import jax
import jax.numpy as jnp
from jax import lax
from jax.experimental import pallas as pl
from jax.experimental.pallas import tpu as pltpu
from jax.experimental.pallas import tpu_sc as plsc

N = 100000
D_IN = 308
D = 100
DP = 128
DPK = DP // 2
NW = 32
WPW = 3200
NP = NW * WPW
CH = 16
NCHUNK = WPW // CH
IPC = CH * 5
NNF = N // 10


def _rne_hi16(u):
    return (u + 0x7FFF + ((u >> 16) & 1)) & jnp.uint32(0xFFFF0000)


def _proj_body(vv_ref, wa_ref, ba_ref, uu_ref):
    x = vv_ref[...].astype(jnp.bfloat16)
    acc = jnp.dot(x, wa_ref[...], preferred_element_type=jnp.float32)
    uu = jnp.maximum(acc + ba_ref[...], 0.0)
    a = lax.bitcast_convert_type(uu[:, :DPK], jnp.uint32)
    b = lax.bitcast_convert_type(uu[:, DPK:], jnp.uint32)
    packed = _rne_hi16(a) | (_rne_hi16(b) >> 16)
    uu_ref[...] = lax.bitcast_convert_type(packed, jnp.float32)


def _project(vv, wa_p, ba_p):
    bn = 1000
    return pl.pallas_call(
        _proj_body,
        grid=(N // bn,),
        in_specs=[
            pl.BlockSpec((bn, D_IN), lambda i: (i, 0)),
            pl.BlockSpec((D_IN, DP), lambda i: (0, 0)),
            pl.BlockSpec((1, DP), lambda i: (0, 0)),
        ],
        out_specs=pl.BlockSpec((bn, DPK), lambda i: (i, 0)),
        out_shape=jax.ShapeDtypeStruct((N, DPK), jnp.float32),
    )(vv, wa_p, ba_p)


CF = 282
CS = 400 - CF
PAIR_W = 6400


def _gather_body(uu, idx_f, padidx, inf, bb, idx_v, inf_v, mask_v, rows0,
                 rows1, bb_buf, sem0, sem1):
    cid = lax.axis_index("c")
    sid = lax.axis_index("s")
    fast = cid == 0
    nch = jnp.where(fast, CF, CS)
    wb = sid * PAIR_W + jnp.where(fast, 0, CF * CH)
    nwords = nch * CH

    last = sid == 15
    real5 = (N - 15 * PAIR_W - 0 * CH) * 5

    @pl.when(fast & ~last)
    def _stage_f():
        pltpu.sync_copy(idx_f.at[pl.ds(wb * 5, CF * IPC)], idx_v)

    @pl.when(fast & last)
    def _stage_f_tail():
        pltpu.sync_copy(idx_f.at[pl.ds(15 * PAIR_W * 5, real5)],
                        idx_v.at[pl.ds(0, real5)])
        pltpu.sync_copy(padidx.at[pl.ds(0, CF * IPC - real5)],
                        idx_v.at[pl.ds(real5, CF * IPC - real5)])

    @pl.when(~fast & ~last)
    def _stage_s():
        pltpu.sync_copy(idx_f.at[pl.ds(wb * 5, CS * IPC)],
                        idx_v.at[pl.ds(0, CS * IPC)])

    @pl.when(~fast & last)
    def _stage_s_tail():
        pltpu.sync_copy(padidx.at[pl.ds(0, CS * IPC)],
                        idx_v.at[pl.ds(0, CS * IPC)])

    pltpu.sync_copy(inf, inf_v)

    ones = jnp.ones((16,), jnp.float32)
    zeros = jnp.zeros((16,), jnp.float32)

    @pl.loop(0, CF)
    def _init_mask(i):
        mask_v[pl.ds(i * 16, 16)] = ones

    @pl.loop(0, NNF // 16)
    def _scatter_mask(i):
        v = inf_v[pl.ds(i * 16, 16)]
        loc = v - wb
        inb = (loc >= 0) & (loc < nwords)
        locc = jnp.clip(loc, 0, CF * CH - 1)
        plsc.store_scatter(mask_v, [locc], zeros, mask=inb)

    def issue(c, rbuf, sem):
        pltpu.async_copy(uu.at[idx_v.at[pl.ds(c * IPC, IPC)]], rbuf, sem)

    def wait(c, rbuf, sem):
        pltpu.make_async_copy(uu.at[idx_v.at[pl.ds(c * IPC, IPC)]], rbuf,
                              sem).wait()

    issue(0, rows0, sem0)
    issue(1, rows1, sem1)

    bf = jnp.bfloat16

    @pl.loop(0, nch, step=2)
    def _chunks(c0):
        for b in range(2):
            c = c0 + b
            rbuf = rows0 if b == 0 else rows1
            sem = sem0 if b == 0 else sem1
            wait(c, rbuf, sem)
            mvv = mask_v[pl.ds(c * CH, 16)]
            for j in range(CH):
                mvf = jnp.full((16,), mvv[j], jnp.float32)
                mv = plsc.pack(mvf, mvf, format=plsc.PackFormat.INTERLEAVED)
                for t in range(DPK // 16):
                    s = pl.ds(t * 16, 16)
                    r = j * 5
                    acc = (plsc.bitcast(rbuf[r, s], bf)
                           + plsc.bitcast(rbuf[r + 1, s], bf)
                           + plsc.bitcast(rbuf[r + 2, s], bf)
                           + plsc.bitcast(rbuf[r + 3, s], bf)
                           + plsc.bitcast(rbuf[r + 4, s], bf))
                    bb_buf[j, s] = plsc.bitcast(acc * mv, jnp.float32)
            pltpu.sync_copy(bb_buf, bb.at[pl.ds(wb + c * CH, CH)])

            @pl.when(c + 2 < nch)
            def _next():
                issue(c + 2, rbuf, sem)


def _gather_sum(uu_packed, idx, padidx, inf):
    mesh = plsc.VectorSubcoreMesh(core_axis_name="c", subcore_axis_name="s",
                                  num_cores=2, num_subcores=16)
    kern = pl.kernel(
        _gather_body,
        out_type=jax.ShapeDtypeStruct((NP, DPK), jnp.float32),
        mesh=mesh,
        compiler_params=pltpu.CompilerParams(needs_layout_passes=False,
                                             use_tc_tiling_on_sc=False),
        scratch_types=[
            pltpu.VMEM((CF * IPC,), jnp.int32),
            pltpu.VMEM((NNF,), jnp.int32),
            pltpu.VMEM((CF * CH,), jnp.float32),
            pltpu.VMEM((IPC, DPK), jnp.float32),
            pltpu.VMEM((IPC, DPK), jnp.float32),
            pltpu.VMEM((CH, DPK), jnp.float32),
            pltpu.SemaphoreType.DMA,
            pltpu.SemaphoreType.DMA,
        ],
    )
    return kern(uu_packed, idx, padidx, inf)


def _mlp_body(bb_ref, w1_ref, b1_ref, wo_ref, bo_ref, w3_ref, st_ref):
    pu = lax.bitcast_convert_type(bb_ref[...], jnp.uint32)
    hi = lax.bitcast_convert_type(pu & jnp.uint32(0xFFFF0000), jnp.float32)
    lo = lax.bitcast_convert_type(pu << 16, jnp.float32)
    bb = jnp.concatenate([hi, lo], axis=1).astype(jnp.bfloat16)
    hid = jnp.tanh(
        jnp.dot(bb, w1_ref[...], preferred_element_type=jnp.float32)
        + b1_ref[...])
    oo = jnp.tanh(
        jnp.dot(hid.astype(jnp.bfloat16), wo_ref[...],
                preferred_element_type=jnp.float32)
        + bo_ref[...])
    st_ref[...] = lax.dot_general(w3_ref[...], oo, (((1,), (1,)), ((), ())),
                                  preferred_element_type=jnp.float32)


def _mlp_scores(bb, w1_p, b1_p, wo_p, bo_p, w3_p):
    bn = 1024
    return pl.pallas_call(
        _mlp_body,
        grid=(NP // bn,),
        in_specs=[
            pl.BlockSpec((bn, DPK), lambda i: (i, 0)),
            pl.BlockSpec((DP, DP), lambda i: (0, 0)),
            pl.BlockSpec((1, DP), lambda i: (0, 0)),
            pl.BlockSpec((DP, DP), lambda i: (0, 0)),
            pl.BlockSpec((1, DP), lambda i: (0, 0)),
            pl.BlockSpec((8, DP), lambda i: (0, 0)),
        ],
        out_specs=pl.BlockSpec((8, bn), lambda i: (0, i)),
        out_shape=jax.ShapeDtypeStruct((8, NP), jnp.float32),
    )(bb, w1_p, b1_p, wo_p, bo_p, w3_p)


def _softmax_body(st_ref, out_ref):
    x = st_ref[...]
    col = lax.broadcasted_iota(jnp.int32, (8, NP), 1)
    valid = col < N
    xm = jnp.where(valid, x, -jnp.inf)
    mx = jnp.max(xm, axis=1, keepdims=True)
    e = jnp.where(valid, jnp.exp(x - mx), 0.0)
    ssum = jnp.sum(e, axis=1, keepdims=True)
    out_ref[...] = e / ssum


def _softmax(st):
    return pl.pallas_call(
        _softmax_body,
        grid=(1,),
        in_specs=[pl.BlockSpec((8, NP), lambda i: (0, 0))],
        out_specs=pl.BlockSpec((8, NP), lambda i: (0, 0)),
        out_shape=jax.ShapeDtypeStruct((8, NP), jnp.float32),
    )(st)


def kernel(indices, indices_not_found, vv, num_words, W_A, b_A, W_b1, b_b1,
           W_bo, b_bo, W_bh, b_bh, W_D, b_D, w_r, b_r, w_c, b_c, w_ce, b_ce):
    bf = jnp.bfloat16
    pad_d = DP - D
    wa_p = jnp.pad(W_A, ((0, 0), (0, pad_d))).astype(bf)
    ba_p = jnp.pad(b_A, (0, pad_d)).reshape(1, DP)
    uu_packed = _project(vv, wa_p, ba_p)

    padidx = jnp.zeros(((NP - N) * 5,), jnp.int32)
    bb = _gather_sum(uu_packed, indices.reshape(N * 5), padidx,
                     indices_not_found)

    w1_p = jnp.pad(W_b1, ((0, pad_d), (0, pad_d))).astype(bf)
    b1_p = jnp.pad(b_b1, (0, pad_d)).reshape(1, DP)
    wo_p = jnp.pad(W_bo, ((0, pad_d), (0, pad_d))).astype(bf)
    bo_p = jnp.pad(b_bo, (0, pad_d)).reshape(1, DP)
    w3 = jnp.stack([w_r, w_c, w_ce]).astype(jnp.float32)
    w3_p = jnp.pad(w3, ((0, 5), (0, pad_d)))
    st = _mlp_scores(bb, w1_p, b1_p, wo_p, bo_p, w3_p)

    sm = _softmax(st)
    return (sm[0, :N], sm[1, :N], sm[2, :N])
